# Initial kernel scaffold; baseline (speedup 1.0000x reference)
import jax
import jax.numpy as jnp
from jax.experimental import pallas as pl


def kernel(input, target, W):
    raise NotImplementedError("write your pallas kernel here")


# speedup vs baseline: 4.6049x; 4.6049x over previous
import functools

import jax
import jax.numpy as jnp
from jax import lax
from jax.experimental import pallas as pl
from jax.experimental.pallas import tpu as pltpu
from jax.experimental.pallas import tpu_sc as plsc

KTOP = 128
BATCH = 1024
IN_DIM = 128
VOCAB = 100000
TV = 2048
NW = 32
ROWS_PER_W = BATCH // NW
HIST_BITS = 10
HIST_B = 1 << HIST_BITS
CAND_CAP = 8192
NVREG_ROW = VOCAB // 16
TAIL_BITS = 32 - HIST_BITS


def _mono_u32(x):
  b = lax.bitcast_convert_type(x, jnp.uint32)
  neg = b >= jnp.uint32(0x80000000)
  return jnp.where(neg, b ^ jnp.uint32(0xFFFFFFFF), b ^ jnp.uint32(0x80000000))


def _unmono_f32(k):
  neg = k < jnp.uint32(0x80000000)
  b = jnp.where(neg, k ^ jnp.uint32(0xFFFFFFFF), k ^ jnp.uint32(0x80000000))
  return lax.bitcast_convert_type(b, jnp.float32)


def _dot(x, w):
  return lax.dot_general(
      x, w, (((1,), (1,)), ((), ())),
      preferred_element_type=jnp.float32,
      precision=lax.Precision.DEFAULT)


def _k1_body(x_ref, w_ref, o_ref):
  o_ref[...] = _dot(x_ref[...], w_ref[...])


def _k1_logits(inp, w):
  grid = pl.cdiv(VOCAB, TV)
  return pl.pallas_call(
      _k1_body,
      grid=(grid,),
      in_specs=[
          pl.BlockSpec((BATCH, IN_DIM), lambda j: (0, 0)),
          pl.BlockSpec((TV, IN_DIM), lambda j: (j, 0)),
      ],
      out_specs=pl.BlockSpec((BATCH, TV), lambda j: (0, j)),
      out_shape=jax.ShapeDtypeStruct((BATCH, VOCAB), jnp.float32),
  )(inp, w)


def _sc_body(logits_hbm, tgt_hbm, tau_hbm, z_hbm, m_hbm,
             row_v, hist_v, cand_v, tgt_v, tau_s, z_s, m_s):
  lane = lax.iota(jnp.int32, 16)
  lane_bank = lane * HIST_B
  ones16 = jnp.ones((16,), jnp.int32)
  zeros16 = jnp.zeros((16,), jnp.int32)

  wid = lax.axis_index("c") * 16 + lax.axis_index("s")
  base = wid * ROWS_PER_W
  pltpu.sync_copy(tgt_hbm.at[pl.ds(base, ROWS_PER_W)], tgt_v)

  def row_loop(r, _):
    row = base + r
    pltpu.sync_copy(logits_hbm.at[row], row_v)

    def zloop(i, c):
      hist_v[pl.ds(i * 16, 16)] = zeros16
      return c
    lax.fori_loop(0, HIST_B * 16 // 16, zloop, 0)

    def p1(i, m_acc):
      v = row_v[pl.ds(i * 16, 16)]
      key = _mono_u32(v)
      d = lax.shift_right_logical(key, jnp.uint32(32 - HIST_BITS))
      addr = d.astype(jnp.int32) + lane_bank
      plsc.addupdate_scatter(hist_v, [addr], ones16)
      return jnp.maximum(m_acc, v)
    m_acc = lax.fori_loop(0, NVREG_ROW, p1,
                          jnp.full((16,), -jnp.inf, jnp.float32))
    m_scal = jnp.max(m_acc)

    def scan_blk(i, carry):
      s_above, bstar = carry
      b0 = HIST_B - 16 * (i + 1)
      tot = zeros16
      for l in range(16):
        tot = tot + hist_v[pl.ds(l * HIST_B + b0, 16)]
      pre = plsc.cumsum(tot)
      blocktot = jnp.max(pre)
      f_gt = s_above + (blocktot - pre)
      f_ge = f_gt + tot
      found = jnp.logical_and(f_gt < KTOP, f_ge >= KTOP)
      cand_b = jnp.max(jnp.where(found, b0 + lane, -1))
      bstar = jnp.maximum(bstar, cand_b)
      return s_above + blocktot, bstar
    _, bstar = lax.fori_loop(0, HIST_B // 16, scan_blk,
                             (jnp.int32(0), jnp.int32(-1)))

    bound = lax.shift_left(bstar.astype(jnp.uint32),
                           jnp.uint32(32 - HIST_BITS))
    def p2(i, off):
      v = row_v[pl.ds(i * 16, 16)]
      key = _mono_u32(v)
      msel = key >= bound
      m01 = jnp.where(msel, ones16, zeros16)
      pos = plsc.cumsum(m01)
      addr = jnp.minimum(off + pos - 1, CAND_CAP - 1)
      plsc.store_scatter(cand_v, [addr], plsc.bitcast(key, jnp.int32),
                         mask=msel)
      return off + plsc.all_reduce_population_count(msel)
    off = lax.fori_loop(0, NVREG_ROW, p2, zeros16)
    c_cand = jnp.minimum(jnp.max(off), CAND_CAP)
    nv = (c_cand + 15) // 16

    def bitloop(kk, p):
      bit = lax.shift_left(jnp.uint32(1), (TAIL_BITS - 1 - kk).astype(jnp.uint32))
      pp = p | bit
      def cnt(i, acc):
        kv = plsc.bitcast(cand_v[pl.ds(i * 16, 16)], jnp.uint32)
        valid = (i * 16 + lane) < c_cand
        ok = jnp.logical_and(valid, kv >= pp)
        return acc + jnp.where(ok, ones16, zeros16)
      acc = lax.fori_loop(0, nv, cnt, zeros16)
      return jnp.where(jnp.sum(acc) >= KTOP, pp, p)
    tau = lax.fori_loop(0, TAIL_BITS, bitloop, bound)

    def zsum(i, acc):
      kv = plsc.bitcast(cand_v[pl.ds(i * 16, 16)], jnp.uint32)
      valid = (i * 16 + lane) < c_cand
      ok = jnp.logical_and(valid, kv >= tau)
      val = _unmono_f32(kv)
      e = jnp.exp(val - m_scal)
      return acc + jnp.where(ok, e, jnp.zeros((16,), jnp.float32))
    zacc = lax.fori_loop(0, nv, zsum, jnp.zeros((16,), jnp.float32))
    z = jnp.sum(zacc)

    rb = lax.shift_left(lax.shift_right_logical(r, 4), 4)
    tvec = tgt_v[pl.ds(rb, 16)]
    t = jnp.max(jnp.where(lane == (r - rb), tvec, -1))
    tb = lax.shift_left(lax.shift_right_logical(t, 4), 4)
    lv = t - tb
    vt = row_v[pl.ds(tb, 16)]
    kt = _mono_u32(vt)
    sel_t = jnp.logical_and(lane == lv, kt < tau)
    z = z + jnp.sum(jnp.where(sel_t, jnp.exp(vt - m_scal),
                              jnp.zeros((16,), jnp.float32)))

    lane0 = lane == 0
    r16 = jnp.full((16,), 0, jnp.int32) + r
    plsc.store_scatter(tau_s, [r16],
                       plsc.bitcast(jnp.full((16,), 0, jnp.uint32) + tau,
                                    jnp.int32), mask=lane0)
    plsc.store_scatter(z_s, [r16], jnp.full((16,), 0.0, jnp.float32) + z,
                       mask=lane0)
    plsc.store_scatter(m_s, [r16], jnp.full((16,), 0.0, jnp.float32) + m_scal,
                       mask=lane0)
    return 0

  lax.fori_loop(0, ROWS_PER_W, row_loop, 0)

  pltpu.sync_copy(tau_s, tau_hbm.at[pl.ds(base, ROWS_PER_W)])
  pltpu.sync_copy(z_s, z_hbm.at[pl.ds(base, ROWS_PER_W)])
  pltpu.sync_copy(m_s, m_hbm.at[pl.ds(base, ROWS_PER_W)])


def _k2_select(logits, target):
  mesh = plsc.VectorSubcoreMesh(core_axis_name="c", subcore_axis_name="s")
  kern = functools.partial(
      pl.kernel,
      out_type=(
          jax.ShapeDtypeStruct((BATCH,), jnp.int32),
          jax.ShapeDtypeStruct((BATCH,), jnp.float32),
          jax.ShapeDtypeStruct((BATCH,), jnp.float32),
      ),
      mesh=mesh,
      compiler_params=pltpu.CompilerParams(needs_layout_passes=False),
      scratch_types=[
          pltpu.VMEM((VOCAB,), jnp.float32),
          pltpu.VMEM((HIST_B * 16,), jnp.int32),
          pltpu.VMEM((CAND_CAP,), jnp.int32),
          pltpu.VMEM((ROWS_PER_W,), jnp.int32),
          pltpu.VMEM((ROWS_PER_W,), jnp.int32),
          pltpu.VMEM((ROWS_PER_W,), jnp.float32),
          pltpu.VMEM((ROWS_PER_W,), jnp.float32),
      ],
  )(_sc_body)
  return kern(logits, target)


def _k3_body(x_ref, w_ref, tau_ref, z_ref, m_ref, tgt_ref, o_ref):
  j = pl.program_id(0)
  logits = _dot(x_ref[...], w_ref[...])
  key = _mono_u32(logits)
  tau = lax.bitcast_convert_type(tau_ref[...], jnp.uint32)
  cols = lax.broadcasted_iota(jnp.int32, (BATCH, TV), 1) + j * TV
  sel = jnp.logical_or(key >= tau, cols == tgt_ref[...])
  zinv = 1.0 / z_ref[...]
  vals = jnp.exp(logits - m_ref[...]) * zinv
  o_ref[...] = jnp.where(sel, vals, jnp.float32(0.0))


def _k3_output(inp, w, tau, z, m, target):
  grid = pl.cdiv(VOCAB, TV)
  col_spec = pl.BlockSpec((BATCH, 1), lambda j: (0, 0))
  return pl.pallas_call(
      _k3_body,
      grid=(grid,),
      in_specs=[
          pl.BlockSpec((BATCH, IN_DIM), lambda j: (0, 0)),
          pl.BlockSpec((TV, IN_DIM), lambda j: (j, 0)),
          col_spec, col_spec, col_spec, col_spec,
      ],
      out_specs=pl.BlockSpec((BATCH, TV), lambda j: (0, j)),
      out_shape=jax.ShapeDtypeStruct((BATCH, VOCAB), jnp.float32),
  )(inp, w, tau.reshape(BATCH, 1), z.reshape(BATCH, 1),
    m.reshape(BATCH, 1), target.reshape(BATCH, 1))


@jax.jit
def kernel(input, target, W):
  logits = _k1_logits(input, W)
  tau, z, m = _k2_select(logits, target)
  return _k3_output(input, W, tau, z, m, target)


# speedup vs baseline: 4.9153x; 1.0674x over previous
import functools

import jax
import jax.numpy as jnp
from jax import lax
from jax.experimental import pallas as pl
from jax.experimental.pallas import tpu as pltpu
from jax.experimental.pallas import tpu_sc as plsc

KTOP = 128
BATCH = 1024
IN_DIM = 128
VOCAB = 100000
TV = 2048
NW = 32
ROWS_PER_W = BATCH // NW
HIST_BITS = 10
HIST_B = 1 << HIST_BITS
CAND_CAP = 8192
NVREG_ROW = VOCAB // 16
UNROLL = 10
TAIL_BITS = 32 - HIST_BITS


def _mono_u32(x):
  b = lax.bitcast_convert_type(x, jnp.uint32)
  neg = b >= jnp.uint32(0x80000000)
  return jnp.where(neg, b ^ jnp.uint32(0xFFFFFFFF), b ^ jnp.uint32(0x80000000))


def _unmono_f32(k):
  neg = k < jnp.uint32(0x80000000)
  b = jnp.where(neg, k ^ jnp.uint32(0xFFFFFFFF), k ^ jnp.uint32(0x80000000))
  return lax.bitcast_convert_type(b, jnp.float32)


def _dot(x, w):
  return lax.dot_general(
      x, w, (((1,), (1,)), ((), ())),
      preferred_element_type=jnp.float32,
      precision=lax.Precision.DEFAULT)


def _k1_body(x_ref, w_ref, o_ref):
  o_ref[...] = _dot(x_ref[...], w_ref[...])


def _k1_logits(inp, w):
  grid = pl.cdiv(VOCAB, TV)
  return pl.pallas_call(
      _k1_body,
      grid=(grid,),
      in_specs=[
          pl.BlockSpec((BATCH, IN_DIM), lambda j: (0, 0)),
          pl.BlockSpec((TV, IN_DIM), lambda j: (j, 0)),
      ],
      out_specs=pl.BlockSpec((BATCH, TV), lambda j: (0, j)),
      out_shape=jax.ShapeDtypeStruct((BATCH, VOCAB), jnp.float32),
  )(inp, w)


def _sc_body(logits_hbm, tgt_hbm, tau_hbm, z_hbm, m_hbm,
             row_v, hist_v, cand_v, tgt_v, tau_s, z_s, m_s):
  lane = lax.iota(jnp.int32, 16)
  lane_bank = lane * HIST_B
  ones16 = jnp.ones((16,), jnp.int32)
  zeros16 = jnp.zeros((16,), jnp.int32)

  wid = lax.axis_index("c") * 16 + lax.axis_index("s")
  base = wid * ROWS_PER_W
  pltpu.sync_copy(tgt_hbm.at[pl.ds(base, ROWS_PER_W)], tgt_v)

  def row_loop(r, _):
    row = base + r
    pltpu.sync_copy(logits_hbm.at[row], row_v)

    def zloop(i, c):
      for u in range(8):
        hist_v[pl.ds(i * 128 + u * 16, 16)] = zeros16
      return c
    lax.fori_loop(0, HIST_B * 16 // 128, zloop, 0)

    def p1(i, m_acc):
      for u in range(UNROLL):
        v = row_v[pl.ds((i * UNROLL + u) * 16, 16)]
        key = _mono_u32(v)
        d = lax.shift_right_logical(key, jnp.uint32(32 - HIST_BITS))
        addr = d.astype(jnp.int32) + lane_bank
        plsc.addupdate_scatter(hist_v, [addr], ones16)
        m_acc = jnp.maximum(m_acc, v)
      return m_acc
    m_acc = lax.fori_loop(0, NVREG_ROW // UNROLL, p1,
                          jnp.full((16,), -jnp.inf, jnp.float32))
    m_scal = jnp.max(m_acc)

    def scan_blk(i, carry):
      s_above, bstar = carry
      b0 = HIST_B - 16 * (i + 1)
      tot = zeros16
      for l in range(16):
        tot = tot + hist_v[pl.ds(l * HIST_B + b0, 16)]
      pre = plsc.cumsum(tot)
      blocktot = jnp.max(pre)
      f_gt = s_above + (blocktot - pre)
      f_ge = f_gt + tot
      found = jnp.logical_and(f_gt < KTOP, f_ge >= KTOP)
      cand_b = jnp.max(jnp.where(found, b0 + lane, -1))
      bstar = jnp.maximum(bstar, cand_b)
      return s_above + blocktot, bstar
    _, bstar = lax.fori_loop(0, HIST_B // 16, scan_blk,
                             (jnp.int32(0), jnp.int32(-1)))

    bound = lax.shift_left(bstar.astype(jnp.uint32),
                           jnp.uint32(32 - HIST_BITS))
    def p2(i, off):
      for u in range(UNROLL):
        v = row_v[pl.ds((i * UNROLL + u) * 16, 16)]
        key = _mono_u32(v)
        msel = key >= bound
        m01 = jnp.where(msel, ones16, zeros16)
        pos = plsc.cumsum(m01)
        addr = jnp.minimum(off + pos - 1, CAND_CAP - 1)
        plsc.store_scatter(cand_v, [addr], plsc.bitcast(key, jnp.int32),
                           mask=msel)
        off = off + plsc.all_reduce_population_count(msel)
      return off
    off = lax.fori_loop(0, NVREG_ROW // UNROLL, p2, zeros16)
    c_cand = jnp.minimum(jnp.max(off), CAND_CAP)
    nv = (c_cand + 15) // 16

    def bitloop(kk, p):
      bit = lax.shift_left(jnp.uint32(1), (TAIL_BITS - 1 - kk).astype(jnp.uint32))
      pp = p | bit
      def cnt(i, acc):
        kv = plsc.bitcast(cand_v[pl.ds(i * 16, 16)], jnp.uint32)
        valid = (i * 16 + lane) < c_cand
        ok = jnp.logical_and(valid, kv >= pp)
        return acc + jnp.where(ok, ones16, zeros16)
      acc = lax.fori_loop(0, nv, cnt, zeros16)
      return jnp.where(jnp.sum(acc) >= KTOP, pp, p)
    tau = lax.fori_loop(0, TAIL_BITS, bitloop, bound)

    def zsum(i, acc):
      kv = plsc.bitcast(cand_v[pl.ds(i * 16, 16)], jnp.uint32)
      valid = (i * 16 + lane) < c_cand
      ok = jnp.logical_and(valid, kv >= tau)
      val = _unmono_f32(kv)
      e = jnp.exp(val - m_scal)
      return acc + jnp.where(ok, e, jnp.zeros((16,), jnp.float32))
    zacc = lax.fori_loop(0, nv, zsum, jnp.zeros((16,), jnp.float32))
    z = jnp.sum(zacc)

    rb = lax.shift_left(lax.shift_right_logical(r, 4), 4)
    tvec = tgt_v[pl.ds(rb, 16)]
    t = jnp.max(jnp.where(lane == (r - rb), tvec, -1))
    tb = lax.shift_left(lax.shift_right_logical(t, 4), 4)
    lv = t - tb
    vt = row_v[pl.ds(tb, 16)]
    kt = _mono_u32(vt)
    sel_t = jnp.logical_and(lane == lv, kt < tau)
    z = z + jnp.sum(jnp.where(sel_t, jnp.exp(vt - m_scal),
                              jnp.zeros((16,), jnp.float32)))

    lane0 = lane == 0
    r16 = jnp.full((16,), 0, jnp.int32) + r
    plsc.store_scatter(tau_s, [r16],
                       plsc.bitcast(jnp.full((16,), 0, jnp.uint32) + tau,
                                    jnp.int32), mask=lane0)
    plsc.store_scatter(z_s, [r16], jnp.full((16,), 0.0, jnp.float32) + z,
                       mask=lane0)
    plsc.store_scatter(m_s, [r16], jnp.full((16,), 0.0, jnp.float32) + m_scal,
                       mask=lane0)
    return 0

  lax.fori_loop(0, ROWS_PER_W, row_loop, 0)

  pltpu.sync_copy(tau_s, tau_hbm.at[pl.ds(base, ROWS_PER_W)])
  pltpu.sync_copy(z_s, z_hbm.at[pl.ds(base, ROWS_PER_W)])
  pltpu.sync_copy(m_s, m_hbm.at[pl.ds(base, ROWS_PER_W)])


def _k2_select(logits, target):
  mesh = plsc.VectorSubcoreMesh(core_axis_name="c", subcore_axis_name="s")
  kern = functools.partial(
      pl.kernel,
      out_type=(
          jax.ShapeDtypeStruct((BATCH,), jnp.int32),
          jax.ShapeDtypeStruct((BATCH,), jnp.float32),
          jax.ShapeDtypeStruct((BATCH,), jnp.float32),
      ),
      mesh=mesh,
      compiler_params=pltpu.CompilerParams(needs_layout_passes=False),
      scratch_types=[
          pltpu.VMEM((VOCAB,), jnp.float32),
          pltpu.VMEM((HIST_B * 16,), jnp.int32),
          pltpu.VMEM((CAND_CAP,), jnp.int32),
          pltpu.VMEM((ROWS_PER_W,), jnp.int32),
          pltpu.VMEM((ROWS_PER_W,), jnp.int32),
          pltpu.VMEM((ROWS_PER_W,), jnp.float32),
          pltpu.VMEM((ROWS_PER_W,), jnp.float32),
      ],
  )(_sc_body)
  return kern(logits, target)


def _k3_body(x_ref, w_ref, tau_ref, z_ref, m_ref, tgt_ref, o_ref):
  j = pl.program_id(0)
  logits = _dot(x_ref[...], w_ref[...])
  key = _mono_u32(logits)
  tau = lax.bitcast_convert_type(tau_ref[...], jnp.uint32)
  cols = lax.broadcasted_iota(jnp.int32, (BATCH, TV), 1) + j * TV
  sel = jnp.logical_or(key >= tau, cols == tgt_ref[...])
  zinv = 1.0 / z_ref[...]
  vals = jnp.exp(logits - m_ref[...]) * zinv
  o_ref[...] = jnp.where(sel, vals, jnp.float32(0.0))


def _k3_output(inp, w, tau, z, m, target):
  grid = pl.cdiv(VOCAB, TV)
  col_spec = pl.BlockSpec((BATCH, 1), lambda j: (0, 0))
  return pl.pallas_call(
      _k3_body,
      grid=(grid,),
      in_specs=[
          pl.BlockSpec((BATCH, IN_DIM), lambda j: (0, 0)),
          pl.BlockSpec((TV, IN_DIM), lambda j: (j, 0)),
          col_spec, col_spec, col_spec, col_spec,
      ],
      out_specs=pl.BlockSpec((BATCH, TV), lambda j: (0, j)),
      out_shape=jax.ShapeDtypeStruct((BATCH, VOCAB), jnp.float32),
  )(inp, w, tau.reshape(BATCH, 1), z.reshape(BATCH, 1),
    m.reshape(BATCH, 1), target.reshape(BATCH, 1))


@jax.jit
def kernel(input, target, W):
  logits = _k1_logits(input, W)
  tau, z, m = _k2_select(logits, target)
  return _k3_output(input, W, tau, z, m, target)
